# baseline (device time: 79797 ns/iter reference)
import functools

import jax
import jax.numpy as jnp
from jax import lax
from jax.experimental import pallas as pl
from jax.experimental.pallas import tpu as pltpu

N_DEV = 4
N_LAYERS = 3
N_HOPS = N_DEV - 1


def kernel(x, Win0, Wout0, Win1, Wout1, Win2, Wout2):
    b, d = x.shape

    def body(x_ref, win0_ref, wout0_ref, win1_ref, wout1_ref,
             win2_ref, wout2_ref, out_ref, comm_ref, send_sems, recv_sems):
        my_pos = lax.axis_index("i")
        left = (my_pos - 1) % N_DEV
        right = (my_pos + 1) % N_DEV

        barrier_sem = pltpu.get_barrier_semaphore()
        for nbr in [left, right]:
            pl.semaphore_signal(
                barrier_sem, inc=1,
                device_id=(nbr,), device_id_type=pl.DeviceIdType.MESH,
            )
        pl.semaphore_wait(barrier_sem, 2)

        def layer(xin, win_ref, wout_ref, li):
            h = jnp.maximum(
                jnp.dot(xin, win_ref[...], preferred_element_type=jnp.float32),
                0.0,
            )
            partial = jnp.dot(
                h, wout_ref[...], preferred_element_type=jnp.float32
            )
            out_ref[...] = partial
            comm_ref[0] = partial
            for hop in range(N_HOPS):
                send_slot = hop % 2
                recv_slot = (hop + 1) % 2
                sem = li * N_HOPS + hop
                rdma = pltpu.make_async_remote_copy(
                    src_ref=comm_ref.at[send_slot],
                    dst_ref=comm_ref.at[recv_slot],
                    send_sem=send_sems.at[sem],
                    recv_sem=recv_sems.at[sem],
                    device_id=(right,),
                    device_id_type=pl.DeviceIdType.MESH,
                )
                rdma.start()
                rdma.wait()
                out_ref[...] += comm_ref[recv_slot]
            return out_ref[...]

        x1 = layer(x_ref[...], win0_ref, wout0_ref, 0)
        x2 = layer(x1, win1_ref, wout1_ref, 1)
        layer(x2, win2_ref, wout2_ref, 2)

        @functools.partial(
            pl.run_scoped, second_barrier=pltpu.SemaphoreType.REGULAR
        )
        def _(second_barrier):
            for nbr in [left, right]:
                pl.semaphore_signal(
                    second_barrier, inc=1,
                    device_id=(nbr,), device_id_type=pl.DeviceIdType.MESH,
                )
            pl.semaphore_wait(second_barrier, 2)

    return pl.pallas_call(
        body,
        out_shape=jax.ShapeDtypeStruct((b, d), jnp.float32),
        in_specs=[pl.BlockSpec(memory_space=pltpu.VMEM)] * 7,
        out_specs=pl.BlockSpec(memory_space=pltpu.VMEM),
        scratch_shapes=[
            pltpu.VMEM((2, b, d), jnp.float32),
            pltpu.SemaphoreType.DMA((N_LAYERS * N_HOPS,)),
            pltpu.SemaphoreType.DMA((N_LAYERS * N_HOPS,)),
        ],
        compiler_params=pltpu.CompilerParams(collective_id=0),
    )(x, Win0, Wout0, Win1, Wout1, Win2, Wout2)


# device time: 43739 ns/iter; 1.8244x vs baseline; 1.8244x over previous
import jax
import jax.numpy as jnp
from jax import lax
from jax.experimental import pallas as pl
from jax.experimental.pallas import tpu as pltpu

N_LAYERS = 3
CHUNKS = 4


def kernel(x, Win0, Wout0, Win1, Wout1, Win2, Wout2):
    b, d = x.shape
    r = b // CHUNKS

    def body(x_ref, win0_ref, wout0_ref, win1_ref, wout1_ref,
             win2_ref, wout2_ref, out_ref,
             pbuf, ybuf, s2buf, xbuf,
             s1_send, s1_recv, s2_send, s2_recv):
        my_pos = lax.axis_index("i")
        py = my_pos ^ 1
        px = 3 - my_pos

        barrier_sem = pltpu.get_barrier_semaphore()
        for nbr in [py, px]:
            pl.semaphore_signal(
                barrier_sem, inc=1,
                device_id=(nbr,), device_id_type=pl.DeviceIdType.MESH,
            )
        pl.semaphore_wait(barrier_sem, 2)

        weights = [(win0_ref, wout0_ref), (win1_ref, wout1_ref),
                   (win2_ref, wout2_ref)]

        xin = x_ref[...]
        for li, (win_ref, wout_ref) in enumerate(weights):
            win = win_ref[...]
            wout = wout_ref[...]
            r1 = []
            for c in range(CHUNKS):
                sl = pl.ds(c * r, r)
                h = jnp.maximum(
                    jnp.dot(xin[c * r:(c + 1) * r, :], win,
                            preferred_element_type=jnp.float32),
                    0.0,
                )
                pbuf[sl, :] = jnp.dot(
                    h, wout, preferred_element_type=jnp.float32
                )
                rdma = pltpu.make_async_remote_copy(
                    src_ref=pbuf.at[sl, :],
                    dst_ref=ybuf.at[li, sl, :],
                    send_sem=s1_send.at[li, c],
                    recv_sem=s1_recv.at[li, c],
                    device_id=(py,),
                    device_id_type=pl.DeviceIdType.MESH,
                )
                rdma.start()
                r1.append(rdma)
            r2 = []
            for c in range(CHUNKS):
                sl = pl.ds(c * r, r)
                r1[c].wait()
                s2buf[sl, :] = pbuf[sl, :] + ybuf[li, sl, :]
                rdma = pltpu.make_async_remote_copy(
                    src_ref=s2buf.at[sl, :],
                    dst_ref=xbuf.at[li, sl, :],
                    send_sem=s2_send.at[li, c],
                    recv_sem=s2_recv.at[li, c],
                    device_id=(px,),
                    device_id_type=pl.DeviceIdType.MESH,
                )
                rdma.start()
                r2.append(rdma)
            for c in range(CHUNKS):
                sl = pl.ds(c * r, r)
                r2[c].wait()
                out_ref[sl, :] = s2buf[sl, :] + xbuf[li, sl, :]
            xin = out_ref[...]

    return pl.pallas_call(
        body,
        out_shape=jax.ShapeDtypeStruct((b, d), jnp.float32),
        in_specs=[pl.BlockSpec(memory_space=pltpu.VMEM)] * 7,
        out_specs=pl.BlockSpec(memory_space=pltpu.VMEM),
        scratch_shapes=[
            pltpu.VMEM((b, d), jnp.float32),
            pltpu.VMEM((N_LAYERS, b, d), jnp.float32),
            pltpu.VMEM((b, d), jnp.float32),
            pltpu.VMEM((N_LAYERS, b, d), jnp.float32),
            pltpu.SemaphoreType.DMA((N_LAYERS, CHUNKS)),
            pltpu.SemaphoreType.DMA((N_LAYERS, CHUNKS)),
            pltpu.SemaphoreType.DMA((N_LAYERS, CHUNKS)),
            pltpu.SemaphoreType.DMA((N_LAYERS, CHUNKS)),
        ],
        compiler_params=pltpu.CompilerParams(collective_id=0),
    )(x, Win0, Wout0, Win1, Wout1, Win2, Wout2)


# device time: 37778 ns/iter; 2.1123x vs baseline; 1.1578x over previous
import jax
import jax.numpy as jnp
from jax import lax
from jax.experimental import pallas as pl
from jax.experimental.pallas import tpu as pltpu

N_LAYERS = 3
CHUNKS = 4


def kernel(x, Win0, Wout0, Win1, Wout1, Win2, Wout2):
    b, d = x.shape
    r = b // CHUNKS

    def body(x_ref, win0_ref, wout0_ref, win1_ref, wout1_ref,
             win2_ref, wout2_ref, out_ref,
             pbuf, ybuf, s2buf, xbuf,
             s1_send, s1_recv, s2_send, s2_recv):
        my_pos = lax.axis_index("i")
        py = my_pos ^ 1
        px = 3 - my_pos

        barrier_sem = pltpu.get_barrier_semaphore()
        for nbr in [py, px]:
            pl.semaphore_signal(
                barrier_sem, inc=1,
                device_id=(nbr,), device_id_type=pl.DeviceIdType.MESH,
            )
        pl.semaphore_wait(barrier_sem, 2)

        weights = [(win0_ref, wout0_ref), (win1_ref, wout1_ref),
                   (win2_ref, wout2_ref)]

        r2_prev = None
        for li, (win_ref, wout_ref) in enumerate(weights):
            win = win_ref[...]
            wout = wout_ref[...]
            r1 = []
            for c in range(CHUNKS):
                sl = pl.ds(c * r, r)
                if li == 0:
                    xc = x_ref[sl, :]
                else:
                    r2_prev[c].wait()
                    xc = s2buf[sl, :] + xbuf[li - 1, sl, :]
                h = jnp.maximum(
                    jnp.dot(xc, win, preferred_element_type=jnp.float32),
                    0.0,
                )
                pbuf[sl, :] = jnp.dot(
                    h, wout, preferred_element_type=jnp.float32
                )
                rdma = pltpu.make_async_remote_copy(
                    src_ref=pbuf.at[sl, :],
                    dst_ref=ybuf.at[li, sl, :],
                    send_sem=s1_send.at[li, c],
                    recv_sem=s1_recv.at[li, c],
                    device_id=(py,),
                    device_id_type=pl.DeviceIdType.MESH,
                )
                rdma.start()
                r1.append(rdma)
            r2 = []
            for c in range(CHUNKS):
                sl = pl.ds(c * r, r)
                r1[c].wait()
                s2buf[sl, :] = pbuf[sl, :] + ybuf[li, sl, :]
                rdma = pltpu.make_async_remote_copy(
                    src_ref=s2buf.at[sl, :],
                    dst_ref=xbuf.at[li, sl, :],
                    send_sem=s2_send.at[li, c],
                    recv_sem=s2_recv.at[li, c],
                    device_id=(px,),
                    device_id_type=pl.DeviceIdType.MESH,
                )
                rdma.start()
                r2.append(rdma)
            r2_prev = r2
        for c in range(CHUNKS):
            sl = pl.ds(c * r, r)
            r2_prev[c].wait()
            out_ref[sl, :] = s2buf[sl, :] + xbuf[N_LAYERS - 1, sl, :]

    return pl.pallas_call(
        body,
        out_shape=jax.ShapeDtypeStruct((b, d), jnp.float32),
        in_specs=[pl.BlockSpec(memory_space=pltpu.VMEM)] * 7,
        out_specs=pl.BlockSpec(memory_space=pltpu.VMEM),
        scratch_shapes=[
            pltpu.VMEM((b, d), jnp.float32),
            pltpu.VMEM((N_LAYERS, b, d), jnp.float32),
            pltpu.VMEM((b, d), jnp.float32),
            pltpu.VMEM((N_LAYERS, b, d), jnp.float32),
            pltpu.SemaphoreType.DMA((N_LAYERS, CHUNKS)),
            pltpu.SemaphoreType.DMA((N_LAYERS, CHUNKS)),
            pltpu.SemaphoreType.DMA((N_LAYERS, CHUNKS)),
            pltpu.SemaphoreType.DMA((N_LAYERS, CHUNKS)),
        ],
        compiler_params=pltpu.CompilerParams(collective_id=0),
    )(x, Win0, Wout0, Win1, Wout1, Win2, Wout2)


# device time: 29194 ns/iter; 2.7333x vs baseline; 1.2940x over previous
import jax
import jax.numpy as jnp
from jax import lax
from jax.experimental import pallas as pl
from jax.experimental.pallas import tpu as pltpu

N_LAYERS = 3
CHUNKS = 4


def kernel(x, Win0, Wout0, Win1, Wout1, Win2, Wout2):
    b, d = x.shape
    r = b // CHUNKS

    def body(x_ref, win0_ref, wout0_ref, win1_ref, wout1_ref,
             win2_ref, wout2_ref, out_ref,
             pbuf, ybuf, s2buf, xbuf,
             s1_send, s1_recv, s2_send, s2_recv):
        my_pos = lax.axis_index("i")
        py = my_pos ^ 1
        px = 3 - my_pos

        barrier_sem = pltpu.get_barrier_semaphore()
        for nbr in [py, px]:
            pl.semaphore_signal(
                barrier_sem, inc=1,
                device_id=(nbr,), device_id_type=pl.DeviceIdType.MESH,
            )
        pl.semaphore_wait(barrier_sem, 2)

        weights = [(win0_ref, wout0_ref), (win1_ref, wout1_ref),
                   (win2_ref, wout2_ref)]

        r2_prev = None
        for li, (win_ref, wout_ref) in enumerate(weights):
            win = win_ref[...].astype(jnp.bfloat16)
            wout = wout_ref[...].astype(jnp.bfloat16)
            r1 = []
            for c in range(CHUNKS):
                sl = pl.ds(c * r, r)
                if li == 0:
                    xc = x_ref[sl, :].astype(jnp.bfloat16)
                else:
                    r2_prev[c].wait()
                    xc = s2buf[sl, :] + xbuf[li - 1, sl, :]
                h = jnp.maximum(
                    jnp.dot(xc, win, preferred_element_type=jnp.float32),
                    0.0,
                ).astype(jnp.bfloat16)
                pbuf[sl, :] = jnp.dot(
                    h, wout, preferred_element_type=jnp.float32
                ).astype(jnp.bfloat16)
                rdma = pltpu.make_async_remote_copy(
                    src_ref=pbuf.at[sl, :],
                    dst_ref=ybuf.at[li, sl, :],
                    send_sem=s1_send.at[li, c],
                    recv_sem=s1_recv.at[li, c],
                    device_id=(py,),
                    device_id_type=pl.DeviceIdType.MESH,
                )
                rdma.start()
                r1.append(rdma)
            r2 = []
            for c in range(CHUNKS):
                sl = pl.ds(c * r, r)
                r1[c].wait()
                s2buf[sl, :] = pbuf[sl, :] + ybuf[li, sl, :]
                rdma = pltpu.make_async_remote_copy(
                    src_ref=s2buf.at[sl, :],
                    dst_ref=xbuf.at[li, sl, :],
                    send_sem=s2_send.at[li, c],
                    recv_sem=s2_recv.at[li, c],
                    device_id=(px,),
                    device_id_type=pl.DeviceIdType.MESH,
                )
                rdma.start()
                r2.append(rdma)
            r2_prev = r2
        for c in range(CHUNKS):
            sl = pl.ds(c * r, r)
            r2_prev[c].wait()
            out_ref[sl, :] = (
                s2buf[sl, :] + xbuf[N_LAYERS - 1, sl, :]
            ).astype(jnp.float32)

    return pl.pallas_call(
        body,
        out_shape=jax.ShapeDtypeStruct((b, d), jnp.float32),
        in_specs=[pl.BlockSpec(memory_space=pltpu.VMEM)] * 7,
        out_specs=pl.BlockSpec(memory_space=pltpu.VMEM),
        scratch_shapes=[
            pltpu.VMEM((b, d), jnp.bfloat16),
            pltpu.VMEM((N_LAYERS, b, d), jnp.bfloat16),
            pltpu.VMEM((b, d), jnp.bfloat16),
            pltpu.VMEM((N_LAYERS, b, d), jnp.bfloat16),
            pltpu.SemaphoreType.DMA((N_LAYERS, CHUNKS)),
            pltpu.SemaphoreType.DMA((N_LAYERS, CHUNKS)),
            pltpu.SemaphoreType.DMA((N_LAYERS, CHUNKS)),
            pltpu.SemaphoreType.DMA((N_LAYERS, CHUNKS)),
        ],
        compiler_params=pltpu.CompilerParams(collective_id=0),
    )(x, Win0, Wout0, Win1, Wout1, Win2, Wout2)
